# baseline (speedup 1.0000x reference)
import functools

import jax
import jax.numpy as jnp
from jax import lax
from jax.experimental import pallas as pl
from jax.experimental.pallas import tpu as pltpu
from jax.experimental.pallas import tpu_sc as plsc

NU = 40000
NI = 60000
NN = NU + NI
D = 32
H = 16
E = 1600000
NT = 16
B = 384
SUB = B // 128
C = 264
EP = NT * C * B
NR128 = EP // 128
R3 = 3
R6 = 6
RPT_A = 6256
RPT_L = NN - 15 * RPT_A
ZB = 128


def _layer_body(table, rows_h, cols_h, w_h, out,
                colb, rowb, wb, gb, zb, acc,
                sl0, sl1, sl2, sl3, sl4, sl5,
                sg0, sg1, sg2, ss0, ss1, ss2, sz):
    c = lax.axis_index("c")
    s = lax.axis_index("s")
    sem_ld = (sl0, sl1, sl2, sl3, sl4, sl5)
    sem_g = (sg0, sg1, sg2)
    sem_s = (ss0, ss1, ss2)
    chunk0 = s * C

    coff = c * NN

    def start_loads(g, b6):
        crow = (chunk0 + g) * SUB
        pltpu.async_copy(rows_h.at[pl.ds(crow, SUB)],
                         rowb.at[pl.ds(b6 * SUB, SUB)], sem_ld[b6])
        pltpu.async_copy(cols_h.at[pl.ds(crow, SUB)],
                         colb.at[pl.ds(b6 * SUB, SUB)], sem_ld[b6])
        pltpu.async_copy(w_h.at[pl.ds(crow, SUB)],
                         wb.at[pl.ds(b6 * SUB, SUB)], sem_ld[b6])

    def wait_loads(b6):
        pltpu.make_async_copy(rows_h.at[pl.ds(0, SUB)],
                              rowb.at[pl.ds(b6 * SUB, SUB)], sem_ld[b6]).wait()
        pltpu.make_async_copy(cols_h.at[pl.ds(0, SUB)],
                              colb.at[pl.ds(b6 * SUB, SUB)], sem_ld[b6]).wait()
        pltpu.make_async_copy(w_h.at[pl.ds(0, SUB)],
                              wb.at[pl.ds(b6 * SUB, SUB)], sem_ld[b6]).wait()
        for k in range(SUB):
            def abody(j16, _, row=b6 * SUB + k):
                colb[row, pl.ds(j16 * 16, 16)] = (
                    colb[row, pl.ds(j16 * 16, 16)] + coff)
                return 0
            lax.fori_loop(0, 8, abody, 0)

    def start_gathers(b3, b6):
        for k in range(SUB):
            pltpu.async_copy(table.at[colb.at[b6 * SUB + k]],
                             gb.at[pl.ds(b3 * B + k * 128, 128)], sem_g[b3])

    def wait_gathers(b3, b6):
        for k in range(SUB):
            pltpu.make_async_copy(table.at[colb.at[b6 * SUB + k]],
                                  gb.at[pl.ds(b3 * B + k * 128, 128)],
                                  sem_g[b3]).wait()

    def scale_scatter(b3, b6):
        for k in range(SUB):
            base = b3 * B + k * 128
            wrow = b6 * SUB + k

            def sbody(j16, _, base=base, wrow=wrow):
                wvec = wb[wrow, pl.ds(j16 * 16, 16)]
                r0 = base + j16 * 16
                for l in range(16):
                    gb[r0 + l, :] = gb[r0 + l, :] * wvec[l]
                return 0

            lax.fori_loop(0, 8, sbody, 0)
            pltpu.async_copy(gb.at[pl.ds(base, 128)],
                             acc.at[rowb.at[wrow]], sem_s[b3], add=True)

    def wait_scatters(b3, b6):
        for k in range(SUB):
            pltpu.make_async_copy(gb.at[pl.ds(b3 * B + k * 128, 128)],
                                  acc.at[rowb.at[b6 * SUB + k]],
                                  sem_s[b3]).wait()

    def zb_body(i, _):
        zb[i, :] = jnp.zeros((H,), jnp.float32)
        return 0

    lax.fori_loop(0, ZB, zb_body, 0)

    zbase = s * RPT_A
    nfull = RPT_A // ZB
    for q in range(nfull):
        pltpu.async_copy(zb, acc.at[pl.ds(zbase + q * ZB, ZB)], sz)
    for q in range(nfull):
        pltpu.make_async_copy(zb, acc.at[pl.ds(zbase + q * ZB, ZB)], sz).wait()

    @pl.when(s < NT - 1)
    def _():
        tail = RPT_A - nfull * ZB
        pltpu.sync_copy(zb.at[pl.ds(0, tail)],
                        acc.at[pl.ds(zbase + nfull * ZB, tail)])

    @pl.when(s == NT - 1)
    def _():
        tail = RPT_L - nfull * ZB
        pltpu.sync_copy(zb.at[pl.ds(0, tail)],
                        acc.at[pl.ds(zbase + nfull * ZB, tail)])

    for b in range(3):
        start_loads(b, b)

    plsc.subcore_barrier()

    def group(tg, _):
        for u in range(6):
            g = tg * 6 + u

            @pl.when(g >= 2)
            def _(u=u):
                wait_scatters((u - 2) % 3, (u - 2) % 6)

            @pl.when(g + 3 < C)
            def _(g=g, u=u):
                start_loads(g + 3, (u + 3) % 6)

            wait_loads(u)
            start_gathers(u % 3, u)

            @pl.when(g >= 1)
            def _(u=u):
                wait_gathers((u - 1) % 3, (u - 1) % 6)
                scale_scatter((u - 1) % 3, (u - 1) % 6)
        return 0

    lax.fori_loop(0, C // 6, group, 0)

    wait_gathers((C - 1) % 3, (C - 1) % 6)
    scale_scatter((C - 1) % 3, (C - 1) % 6)
    wait_scatters((C - 2) % 3, (C - 2) % 6)
    wait_scatters((C - 1) % 3, (C - 1) % 6)

    plsc.subcore_barrier()

    @pl.when(s < NT - 1)
    def _():
        pltpu.sync_copy(acc.at[pl.ds(zbase, RPT_A)],
                        out.at[pl.ds(c * NN + zbase, RPT_A)])

    @pl.when(s == NT - 1)
    def _():
        pltpu.sync_copy(acc.at[pl.ds(zbase, RPT_L)],
                        out.at[pl.ds(c * NN + zbase, RPT_L)])


_MESH = plsc.VectorSubcoreMesh(core_axis_name="c", subcore_axis_name="s")

_layer = functools.partial(
    pl.kernel,
    out_type=jax.ShapeDtypeStruct((2 * NN, H), jnp.float32),
    mesh=_MESH,
    compiler_params=pltpu.CompilerParams(use_tc_tiling_on_sc=False),
    scratch_types=[
        pltpu.VMEM((R6 * SUB, 128), jnp.int32),
        pltpu.VMEM((R6 * SUB, 128), jnp.int32),
        pltpu.VMEM((R6 * SUB, 128), jnp.float32),
        pltpu.VMEM((R3 * B, H), jnp.float32),
        pltpu.VMEM((ZB, H), jnp.float32),
        pltpu.VMEM_SHARED((NN, H), jnp.float32),
        pltpu.SemaphoreType.DMA,
        pltpu.SemaphoreType.DMA,
        pltpu.SemaphoreType.DMA,
        pltpu.SemaphoreType.DMA,
        pltpu.SemaphoreType.DMA,
        pltpu.SemaphoreType.DMA,
        pltpu.SemaphoreType.DMA,
        pltpu.SemaphoreType.DMA,
        pltpu.SemaphoreType.DMA,
        pltpu.SemaphoreType.DMA,
        pltpu.SemaphoreType.DMA,
        pltpu.SemaphoreType.DMA,
        pltpu.SemaphoreType.DMA,
    ],
)(_layer_body)


def _mean_body(a_ref, b_ref, c_ref, d_ref, o_ref):
    o_ref[...] = (a_ref[...] + b_ref[...] + c_ref[...] + d_ref[...]) * 0.25


_mean4 = pl.pallas_call(
    _mean_body,
    grid=(25,),
    in_specs=[pl.BlockSpec((1000, 128), lambda i: (i, 0))] * 4,
    out_specs=pl.BlockSpec((1000, 128), lambda i: (i, 0)),
    out_shape=jax.ShapeDtypeStruct((25000, 128), jnp.float32),
)


def kernel(edge_index, edge_weight, user_emb, item_emb):
    rows = edge_index[0].astype(jnp.int32)
    cols = edge_index[1].astype(jnp.int32)
    w = edge_weight.astype(jnp.float32)

    padr = NR128 - E // 128
    rows_p = jnp.pad(rows.reshape(E // 128, 128), ((0, padr), (0, 0)))
    cols_p = jnp.pad(cols.reshape(E // 128, 128), ((0, padr), (0, 0)))
    w_p = jnp.pad(w.reshape(E // 128, 128), ((0, padr), (0, 0)))

    all_emb = jnp.concatenate([user_emb, item_emb], axis=0)
    e0 = jnp.concatenate([all_emb[:, :H], all_emb[:, H:]], axis=0)

    e1 = _layer(e0, rows_p, cols_p, w_p)
    e2 = _layer(e1, rows_p, cols_p, w_p)
    e3 = _layer(e2, rows_p, cols_p, w_p)

    fs = _mean4(e0.reshape(25000, 128), e1.reshape(25000, 128),
                e2.reshape(25000, 128), e3.reshape(25000, 128))
    fs = fs.reshape(2 * NN, H)
    users = jnp.concatenate([fs[:NU], fs[NN:NN + NU]], axis=1)
    items = jnp.concatenate([fs[NU:NN], fs[NN + NU:]], axis=1)
    return (users, items)


# speedup vs baseline: 1.0140x; 1.0140x over previous
import functools

import jax
import jax.numpy as jnp
from jax import lax
from jax.experimental import pallas as pl
from jax.experimental.pallas import tpu as pltpu
from jax.experimental.pallas import tpu_sc as plsc

NU = 40000
NI = 60000
NN = NU + NI
D = 32
H = 16
E = 1600000
NT = 16
B = 384
SUB = B // 128
C = 264
EP = NT * C * B
NR128 = EP // 128
R3 = 3
R6 = 6
RPT_A = 6256
RPT_L = NN - 15 * RPT_A
ZB = 128


def _layer_body(table, rows_h, cols_h, w_h, out,
                colb, rowb, wb, gb, zb, acc,
                sl0, sl1, sl2, sl3, sl4, sl5,
                sg0, sg1, sg2, ss0, ss1, ss2, sz):
    c = lax.axis_index("c")
    s = lax.axis_index("s")
    sem_ld = (sl0, sl1, sl2, sl3, sl4, sl5)
    sem_g = (sg0, sg1, sg2)
    sem_s = (ss0, ss1, ss2)
    chunk0 = s * C

    def start_loads(g, b6):
        crow = (chunk0 + g) * SUB
        pltpu.async_copy(rows_h.at[pl.ds(crow, SUB)],
                         rowb.at[pl.ds(b6 * SUB, SUB)], sem_ld[b6])
        pltpu.async_copy(cols_h.at[pl.ds(crow, SUB)],
                         colb.at[pl.ds(b6 * SUB, SUB)], sem_ld[b6])
        pltpu.async_copy(w_h.at[pl.ds(crow, SUB)],
                         wb.at[pl.ds(b6 * SUB, SUB)], sem_ld[b6])

    def wait_loads(b6):
        pltpu.make_async_copy(rows_h.at[pl.ds(0, SUB)],
                              rowb.at[pl.ds(b6 * SUB, SUB)], sem_ld[b6]).wait()
        pltpu.make_async_copy(cols_h.at[pl.ds(0, SUB)],
                              colb.at[pl.ds(b6 * SUB, SUB)], sem_ld[b6]).wait()
        pltpu.make_async_copy(w_h.at[pl.ds(0, SUB)],
                              wb.at[pl.ds(b6 * SUB, SUB)], sem_ld[b6]).wait()

    def start_gathers(b3, b6):
        @pl.when(c == 0)
        def _():
            for k in range(SUB):
                pltpu.async_copy(table.at[0].at[colb.at[b6 * SUB + k]],
                                 gb.at[pl.ds(b3 * B + k * 128, 128)],
                                 sem_g[b3])

        @pl.when(c == 1)
        def _():
            for k in range(SUB):
                pltpu.async_copy(table.at[1].at[colb.at[b6 * SUB + k]],
                                 gb.at[pl.ds(b3 * B + k * 128, 128)],
                                 sem_g[b3])

    def wait_gathers(b3, b6):
        for k in range(SUB):
            pltpu.make_async_copy(table.at[0].at[colb.at[b6 * SUB + k]],
                                  gb.at[pl.ds(b3 * B + k * 128, 128)],
                                  sem_g[b3]).wait()

    def scale_scatter(b3, b6):
        for k in range(SUB):
            base = b3 * B + k * 128
            wrow = b6 * SUB + k

            def sbody(j16, _, base=base, wrow=wrow):
                wvec = wb[wrow, pl.ds(j16 * 16, 16)]
                r0 = base + j16 * 16
                for l in range(16):
                    gb[r0 + l, :] = gb[r0 + l, :] * wvec[l]
                return 0

            lax.fori_loop(0, 8, sbody, 0)
            pltpu.async_copy(gb.at[pl.ds(base, 128)],
                             acc.at[rowb.at[wrow]], sem_s[b3], add=True)

    def wait_scatters(b3, b6):
        for k in range(SUB):
            pltpu.make_async_copy(gb.at[pl.ds(b3 * B + k * 128, 128)],
                                  acc.at[rowb.at[b6 * SUB + k]],
                                  sem_s[b3]).wait()

    def zb_body(i, _):
        zb[i, :] = jnp.zeros((H,), jnp.float32)
        return 0

    lax.fori_loop(0, ZB, zb_body, 0)

    zbase = s * RPT_A
    nfull = RPT_A // ZB
    for q in range(nfull):
        pltpu.async_copy(zb, acc.at[pl.ds(zbase + q * ZB, ZB)], sz)
    for q in range(nfull):
        pltpu.make_async_copy(zb, acc.at[pl.ds(zbase + q * ZB, ZB)], sz).wait()

    @pl.when(s < NT - 1)
    def _():
        tail = RPT_A - nfull * ZB
        pltpu.sync_copy(zb.at[pl.ds(0, tail)],
                        acc.at[pl.ds(zbase + nfull * ZB, tail)])

    @pl.when(s == NT - 1)
    def _():
        tail = RPT_L - nfull * ZB
        pltpu.sync_copy(zb.at[pl.ds(0, tail)],
                        acc.at[pl.ds(zbase + nfull * ZB, tail)])

    for b in range(3):
        start_loads(b, b)

    plsc.subcore_barrier()

    def group(tg, _):
        for u in range(6):
            g = tg * 6 + u

            @pl.when(g >= 2)
            def _(u=u):
                wait_scatters((u - 2) % 3, (u - 2) % 6)

            @pl.when(g + 3 < C)
            def _(g=g, u=u):
                start_loads(g + 3, (u + 3) % 6)

            wait_loads(u)
            start_gathers(u % 3, u)

            @pl.when(g >= 1)
            def _(u=u):
                wait_gathers((u - 1) % 3, (u - 1) % 6)
                scale_scatter((u - 1) % 3, (u - 1) % 6)
        return 0

    lax.fori_loop(0, C // 6, group, 0)

    wait_gathers((C - 1) % 3, (C - 1) % 6)
    scale_scatter((C - 1) % 3, (C - 1) % 6)
    wait_scatters((C - 2) % 3, (C - 2) % 6)
    wait_scatters((C - 1) % 3, (C - 1) % 6)

    plsc.subcore_barrier()

    @pl.when(s < NT - 1)
    def _():
        pltpu.sync_copy(acc.at[pl.ds(zbase, RPT_A)],
                        out.at[c, pl.ds(zbase, RPT_A)])

    @pl.when(s == NT - 1)
    def _():
        pltpu.sync_copy(acc.at[pl.ds(zbase, RPT_L)],
                        out.at[c, pl.ds(zbase, RPT_L)])


_MESH = plsc.VectorSubcoreMesh(core_axis_name="c", subcore_axis_name="s")

_layer = functools.partial(
    pl.kernel,
    out_type=jax.ShapeDtypeStruct((2, NN, H), jnp.float32),
    mesh=_MESH,
    compiler_params=pltpu.CompilerParams(use_tc_tiling_on_sc=False),
    scratch_types=[
        pltpu.VMEM((R6 * SUB, 128), jnp.int32),
        pltpu.VMEM((R6 * SUB, 128), jnp.int32),
        pltpu.VMEM((R6 * SUB, 128), jnp.float32),
        pltpu.VMEM((R3 * B, H), jnp.float32),
        pltpu.VMEM((ZB, H), jnp.float32),
        pltpu.VMEM_SHARED((NN, H), jnp.float32),
        pltpu.SemaphoreType.DMA,
        pltpu.SemaphoreType.DMA,
        pltpu.SemaphoreType.DMA,
        pltpu.SemaphoreType.DMA,
        pltpu.SemaphoreType.DMA,
        pltpu.SemaphoreType.DMA,
        pltpu.SemaphoreType.DMA,
        pltpu.SemaphoreType.DMA,
        pltpu.SemaphoreType.DMA,
        pltpu.SemaphoreType.DMA,
        pltpu.SemaphoreType.DMA,
        pltpu.SemaphoreType.DMA,
        pltpu.SemaphoreType.DMA,
    ],
)(_layer_body)


def _mean_body(a_ref, b_ref, c_ref, d_ref, o_ref):
    o_ref[...] = (a_ref[...] + b_ref[...] + c_ref[...] + d_ref[...]) * 0.25


_mean4 = pl.pallas_call(
    _mean_body,
    grid=(25,),
    in_specs=[pl.BlockSpec((1000, 128), lambda i: (i, 0))] * 4,
    out_specs=pl.BlockSpec((1000, 128), lambda i: (i, 0)),
    out_shape=jax.ShapeDtypeStruct((25000, 128), jnp.float32),
)


def kernel(edge_index, edge_weight, user_emb, item_emb):
    rows = edge_index[0].astype(jnp.int32)
    cols = edge_index[1].astype(jnp.int32)
    w = edge_weight.astype(jnp.float32)

    padr = NR128 - E // 128
    rows_p = jnp.pad(rows.reshape(E // 128, 128), ((0, padr), (0, 0)))
    cols_p = jnp.pad(cols.reshape(E // 128, 128), ((0, padr), (0, 0)))
    w_p = jnp.pad(w.reshape(E // 128, 128), ((0, padr), (0, 0)))

    all_emb = jnp.concatenate([user_emb, item_emb], axis=0)
    e0 = jnp.stack([all_emb[:, :H], all_emb[:, H:]], axis=0)

    e1 = _layer(e0, rows_p, cols_p, w_p)
    e2 = _layer(e1, rows_p, cols_p, w_p)
    e3 = _layer(e2, rows_p, cols_p, w_p)

    fs = _mean4(e0.reshape(25000, 128), e1.reshape(25000, 128),
                e2.reshape(25000, 128), e3.reshape(25000, 128))
    fs = fs.reshape(2 * NN, H)
    users = jnp.concatenate([fs[:NU], fs[NN:NN + NU]], axis=1)
    items = jnp.concatenate([fs[NU:NN], fs[NN + NU:]], axis=1)
    return (users, items)


# speedup vs baseline: 1.3332x; 1.3148x over previous
import functools

import jax
import jax.numpy as jnp
from jax import lax
from jax.experimental import pallas as pl
from jax.experimental.pallas import tpu as pltpu
from jax.experimental.pallas import tpu_sc as plsc

NU = 40000
NI = 60000
NN = NU + NI
D = 32
H = 16
E = 1600000
NT = 16
B = 384
SUB = B // 128
C = 264
EP = NT * C * B
NR128 = EP // 128
R3 = 3
R6 = 6
RPT_A = 6256
RPT_L = NN - 15 * RPT_A
ZB = 128


def _layer_body(table, rows_h, cols_h, w_h, out,
                colb, rowb, wb, gb, zb, acc,
                sl0, sl1, sl2, sl3, sl4, sl5,
                sg0, sg1, sg2, ss0, ss1, ss2, sz):
    c = lax.axis_index("c")
    s = lax.axis_index("s")
    sem_ld = (sl0, sl1, sl2, sl3, sl4, sl5)
    sem_g = (sg0, sg1, sg2)
    sem_s = (ss0, ss1, ss2)
    chunk0 = s * C

    def start_loads(g, b6):
        crow = (chunk0 + g) * SUB
        pltpu.async_copy(rows_h.at[pl.ds(crow, SUB)],
                         rowb.at[pl.ds(b6 * SUB, SUB)], sem_ld[b6])
        pltpu.async_copy(cols_h.at[pl.ds(crow, SUB)],
                         colb.at[pl.ds(b6 * SUB, SUB)], sem_ld[b6])
        pltpu.async_copy(w_h.at[pl.ds(crow, SUB)],
                         wb.at[pl.ds(b6 * SUB, SUB)], sem_ld[b6])

    def wait_loads(b6):
        pltpu.make_async_copy(rows_h.at[pl.ds(0, SUB)],
                              rowb.at[pl.ds(b6 * SUB, SUB)], sem_ld[b6]).wait()
        pltpu.make_async_copy(cols_h.at[pl.ds(0, SUB)],
                              colb.at[pl.ds(b6 * SUB, SUB)], sem_ld[b6]).wait()
        pltpu.make_async_copy(w_h.at[pl.ds(0, SUB)],
                              wb.at[pl.ds(b6 * SUB, SUB)], sem_ld[b6]).wait()

    def start_gathers(b3, b6):
        @pl.when(c == 0)
        def _():
            for k in range(SUB):
                pltpu.async_copy(table.at[0].at[colb.at[b6 * SUB + k]],
                                 gb.at[pl.ds(b3 * B + k * 128, 128)],
                                 sem_g[b3])

        @pl.when(c == 1)
        def _():
            for k in range(SUB):
                pltpu.async_copy(table.at[1].at[colb.at[b6 * SUB + k]],
                                 gb.at[pl.ds(b3 * B + k * 128, 128)],
                                 sem_g[b3])

    def wait_gathers(b3, b6):
        for k in range(SUB):
            pltpu.make_async_copy(table.at[0].at[colb.at[b6 * SUB + k]],
                                  gb.at[pl.ds(b3 * B + k * 128, 128)],
                                  sem_g[b3]).wait()

    def scale_scatter(b3, b6):
        for k in range(SUB):
            base = b3 * B + k * 128
            wrow = b6 * SUB + k

            def sbody(j16, _, base=base, wrow=wrow):
                wvec = wb[wrow, pl.ds(j16 * 16, 16)]
                r0 = base + j16 * 16
                for l in range(16):
                    gb[r0 + l, :] = gb[r0 + l, :] * wvec[l]
                return 0

            lax.fori_loop(0, 8, sbody, 0)
            pltpu.async_copy(gb.at[pl.ds(base, 128)],
                             acc.at[rowb.at[wrow]], sem_s[b3], add=True)

    def wait_scatters(b3, b6):
        for k in range(SUB):
            pltpu.make_async_copy(gb.at[pl.ds(b3 * B + k * 128, 128)],
                                  acc.at[rowb.at[b6 * SUB + k]],
                                  sem_s[b3]).wait()

    def zb_body(i, _):
        zb[i, :] = jnp.zeros((H,), jnp.float32)
        return 0

    lax.fori_loop(0, ZB, zb_body, 0)

    zbase = s * RPT_A
    nfull = RPT_A // ZB
    for q in range(nfull):
        pltpu.async_copy(zb, acc.at[pl.ds(zbase + q * ZB, ZB)], sz)
    for q in range(nfull):
        pltpu.make_async_copy(zb, acc.at[pl.ds(zbase + q * ZB, ZB)], sz).wait()

    @pl.when(s < NT - 1)
    def _():
        tail = RPT_A - nfull * ZB
        pltpu.sync_copy(zb.at[pl.ds(0, tail)],
                        acc.at[pl.ds(zbase + nfull * ZB, tail)])

    @pl.when(s == NT - 1)
    def _():
        tail = RPT_L - nfull * ZB
        pltpu.sync_copy(zb.at[pl.ds(0, tail)],
                        acc.at[pl.ds(zbase + nfull * ZB, tail)])

    for b in range(3):
        start_loads(b, b)

    plsc.subcore_barrier()

    def group(tg, _):
        for u in range(6):
            g = tg * 6 + u

            @pl.when(g >= 2)
            def _(u=u):
                wait_scatters((u - 2) % 3, (u - 2) % 6)

            @pl.when(g + 3 < C)
            def _(g=g, u=u):
                start_loads(g + 3, (u + 3) % 6)

            wait_loads(u)
            start_gathers(u % 3, u)

            @pl.when(g >= 1)
            def _(u=u):
                wait_gathers((u - 1) % 3, (u - 1) % 6)
                scale_scatter((u - 1) % 3, (u - 1) % 6)
        return 0

    lax.fori_loop(0, C // 6, group, 0)

    wait_gathers((C - 1) % 3, (C - 1) % 6)
    scale_scatter((C - 1) % 3, (C - 1) % 6)
    wait_scatters((C - 2) % 3, (C - 2) % 6)
    wait_scatters((C - 1) % 3, (C - 1) % 6)

    plsc.subcore_barrier()

    @pl.when(s < NT - 1)
    def _():
        pltpu.sync_copy(acc.at[pl.ds(zbase, RPT_A)],
                        out.at[c, pl.ds(zbase, RPT_A)])

    @pl.when(s == NT - 1)
    def _():
        pltpu.sync_copy(acc.at[pl.ds(zbase, RPT_L)],
                        out.at[c, pl.ds(zbase, RPT_L)])


_MESH = plsc.VectorSubcoreMesh(core_axis_name="c", subcore_axis_name="s")

_layer = functools.partial(
    pl.kernel,
    out_type=jax.ShapeDtypeStruct((2, NN, H), jnp.float32),
    mesh=_MESH,
    compiler_params=pltpu.CompilerParams(use_tc_tiling_on_sc=False),
    scratch_types=[
        pltpu.VMEM((R6 * SUB, 128), jnp.int32),
        pltpu.VMEM((R6 * SUB, 128), jnp.int32),
        pltpu.VMEM((R6 * SUB, 128), jnp.float32),
        pltpu.VMEM((R3 * B, H), jnp.float32),
        pltpu.VMEM((ZB, H), jnp.float32),
        pltpu.VMEM_SHARED((NN, H), jnp.float32),
        pltpu.SemaphoreType.DMA,
        pltpu.SemaphoreType.DMA,
        pltpu.SemaphoreType.DMA,
        pltpu.SemaphoreType.DMA,
        pltpu.SemaphoreType.DMA,
        pltpu.SemaphoreType.DMA,
        pltpu.SemaphoreType.DMA,
        pltpu.SemaphoreType.DMA,
        pltpu.SemaphoreType.DMA,
        pltpu.SemaphoreType.DMA,
        pltpu.SemaphoreType.DMA,
        pltpu.SemaphoreType.DMA,
        pltpu.SemaphoreType.DMA,
    ],
)(_layer_body)


def _mean_body(a_ref, b_ref, c_ref, d_ref, o_ref):
    o_ref[...] = (a_ref[...] + b_ref[...] + c_ref[...] + d_ref[...]) * 0.25


_mean4 = pl.pallas_call(
    _mean_body,
    grid=(25,),
    in_specs=[pl.BlockSpec((1000, 128), lambda i: (i, 0))] * 4,
    out_specs=pl.BlockSpec((1000, 128), lambda i: (i, 0)),
    out_shape=jax.ShapeDtypeStruct((25000, 128), jnp.float32),
)


def kernel(edge_index, edge_weight, user_emb, item_emb):
    rows = edge_index[0].astype(jnp.int32)
    cols = edge_index[1].astype(jnp.int32)
    w = edge_weight.astype(jnp.float32)

    padr = NR128 - E // 128
    pidx = (jnp.arange(padr * 128, dtype=jnp.int32) % NN).reshape(padr, 128)
    rows_p = jnp.concatenate([rows.reshape(E // 128, 128), pidx], axis=0)
    cols_p = jnp.concatenate([cols.reshape(E // 128, 128), pidx], axis=0)
    w_p = jnp.pad(w.reshape(E // 128, 128), ((0, padr), (0, 0)))

    all_emb = jnp.concatenate([user_emb, item_emb], axis=0)
    e0 = jnp.stack([all_emb[:, :H], all_emb[:, H:]], axis=0)

    e1 = _layer(e0, rows_p, cols_p, w_p)
    e2 = _layer(e1, rows_p, cols_p, w_p)
    e3 = _layer(e2, rows_p, cols_p, w_p)

    fs = _mean4(e0.reshape(25000, 128), e1.reshape(25000, 128),
                e2.reshape(25000, 128), e3.reshape(25000, 128))
    fs = fs.reshape(2 * NN, H)
    users = jnp.concatenate([fs[:NU], fs[NN:NN + NU]], axis=1)
    items = jnp.concatenate([fs[NU:NN], fs[NN + NU:]], axis=1)
    return (users, items)
